# Initial kernel scaffold; baseline (speedup 1.0000x reference)
import jax
import jax.numpy as jnp
from jax.experimental import pallas as pl


def kernel(x, edge_index, W1, b1, W2, b2):
    raise NotImplementedError("write your pallas kernel here")


# speedup vs baseline: 12.4929x; 12.4929x over previous
import functools

import jax
import jax.numpy as jnp
from jax import lax
from jax.experimental import pallas as pl
from jax.experimental.pallas import tpu as pltpu
from jax.experimental.pallas import tpu_sc as plsc

N = 10000
D = 128
E = 320000
NC, NS = 2, 16
NW = NC * NS
C = 125
ERows = E // C
NCH_D = E // (NW * C)
NCH_S = E // (NS * C)
NP = 10240
RPT = NP // NS
HALF = NP // 2
ACC_R = HALF + 8
SPT = HALF // NS
DW = 16
RB = 1000
NB = N // RB


def _wid():
    return lax.axis_index("c") * NS + lax.axis_index("s")


def _deg_body(dsth_hbm, ones_hbm, zeros_hbm, degp_hbm, didx, ones_v, acc):
    cid = lax.axis_index("c")
    sid = lax.axis_index("s")
    pltpu.sync_copy(zeros_hbm, acc.at[pl.ds(sid * SPT, SPT)])
    pltpu.sync_copy(dsth_hbm.at[cid, pl.ds(sid * NCH_S, NCH_S)], didx)
    pltpu.sync_copy(ones_hbm, ones_v)
    plsc.subcore_barrier()

    def body(c, _):
        pltpu.sync_copy(ones_v, acc.at[didx.at[c]], add=True)
        return 0

    lax.fori_loop(0, NCH_S, body, 0)
    plsc.subcore_barrier()
    pltpu.sync_copy(acc.at[pl.ds(sid * SPT, SPT)],
                    degp_hbm.at[cid, pl.ds(sid * SPT, SPT)])


def _segsum_body(y_hbm, src_hbm, dsth_hbm, zeros_hbm, part_hbm,
                 sidx, didx, buf0, buf1, acc, sem0, sem1, szero):
    cid = lax.axis_index("c")
    sid = lax.axis_index("s")
    dz = pltpu.async_copy(zeros_hbm, acc.at[pl.ds(sid * SPT, SPT)], szero)
    ds_ = pltpu.async_copy(src_hbm.at[pl.ds(sid * NCH_S, NCH_S)], sidx, sem0)
    dd = pltpu.async_copy(dsth_hbm.at[cid, pl.ds(sid * NCH_S, NCH_S)], didx,
                          sem1)
    dz.wait()
    ds_.wait()
    dd.wait()
    plsc.subcore_barrier()

    pltpu.async_copy(y_hbm.at[sidx.at[0]], buf0, sem0)

    def body(i, _):
        c0 = 2 * i
        pltpu.make_async_copy(y_hbm.at[sidx.at[c0]], buf0, sem0).wait()
        pltpu.async_copy(y_hbm.at[sidx.at[c0 + 1]], buf1, sem1)
        pltpu.sync_copy(buf0, acc.at[didx.at[c0]], add=True)
        pltpu.make_async_copy(y_hbm.at[sidx.at[c0 + 1]], buf1, sem1).wait()

        @pl.when(c0 + 2 < NCH_S)
        def _():
            pltpu.async_copy(y_hbm.at[sidx.at[c0 + 2]], buf0, sem0)

        pltpu.sync_copy(buf1, acc.at[didx.at[c0 + 1]], add=True)
        return 0

    lax.fori_loop(0, NCH_S // 2, body, 0)
    plsc.subcore_barrier()
    pltpu.sync_copy(acc.at[pl.ds(sid * SPT, SPT)],
                    part_hbm.at[cid, pl.ds(sid * SPT, SPT)])


@functools.cache
def _sc_kernels():
    mesh = plsc.VectorSubcoreMesh(core_axis_name="c", subcore_axis_name="s",
                                  num_cores=NC, num_subcores=NS)
    deg = pl.kernel(
        _deg_body,
        out_type=jax.ShapeDtypeStruct((NC, HALF, D), jnp.float32),
        mesh=mesh,
        scratch_types=[
            pltpu.VMEM((NCH_S, C), jnp.int32),
            pltpu.VMEM((C, D), jnp.float32),
            pltpu.VMEM_SHARED((ACC_R, D), jnp.float32),
        ],
    )
    segsum = pl.kernel(
        _segsum_body,
        out_type=jax.ShapeDtypeStruct((NC, HALF, D), jnp.float32),
        mesh=mesh,
        scratch_types=[
            pltpu.VMEM((NCH_S, C), jnp.int32),
            pltpu.VMEM((NCH_S, C), jnp.int32),
            pltpu.VMEM((C, D), jnp.float32),
            pltpu.VMEM((C, D), jnp.float32),
            pltpu.VMEM_SHARED((ACC_R, D), jnp.float32),
            pltpu.SemaphoreType.DMA,
            pltpu.SemaphoreType.DMA,
            pltpu.SemaphoreType.DMA,
        ],
    )
    return deg, segsum


def _dsth_body(dst_ref, out_ref):
    d = dst_ref[...]
    out_ref[0] = jnp.where(d < HALF, d, HALF)
    out_ref[1] = jnp.where(d >= HALF, d - HALF, HALF)


def _dinv(deg_ref):
    return lax.rsqrt(deg_ref[:, :1] + 1.0)


def _y1_body(x_ref, w_ref, deg_ref, y_ref):
    y_ref[...] = _dinv(deg_ref) * jnp.dot(
        x_ref[...], w_ref[...], preferred_element_type=jnp.float32)


def _mid_body(p_ref, y1_ref, deg_ref, b1_ref, w2_ref, y2_ref):
    dinv = _dinv(deg_ref)
    h = jnp.maximum(dinv * (p_ref[...] + y1_ref[...]) + b1_ref[...], 0.0)
    y2_ref[...] = dinv * jnp.dot(h, w2_ref[...],
                                 preferred_element_type=jnp.float32)


def _out_body(p_ref, y2_ref, deg_ref, b2_ref, o_ref):
    dinv = _dinv(deg_ref)
    o_ref[...] = dinv * (p_ref[...] + y2_ref[...]) + b2_ref[...]


_row_spec = pl.BlockSpec((RB, D), lambda i: (i, 0))
_full_spec = pl.BlockSpec((D, D), lambda i: (0, 0))
_bias_spec = pl.BlockSpec((1, D), lambda i: (0, 0))
_grid = (NB,)
_nd_f32 = jax.ShapeDtypeStruct((N, D), jnp.float32)


def kernel(x, edge_index, W1, b1, W2, b2):
    src2 = edge_index[0].astype(jnp.int32).reshape(ERows, C)
    dst2 = edge_index[1].astype(jnp.int32).reshape(ERows, C)
    ones_c = jnp.ones((C, D), jnp.float32)
    zeros_rd = jnp.zeros((SPT, D), jnp.float32)
    b1r = b1.reshape(1, D)
    b2r = b2.reshape(1, D)

    dsth = pl.pallas_call(
        _dsth_body,
        grid=(ERows // 320,),
        in_specs=[pl.BlockSpec((320, C), lambda i: (i, 0))],
        out_specs=pl.BlockSpec((NC, 320, C), lambda i: (0, i, 0)),
        out_shape=jax.ShapeDtypeStruct((NC, ERows, C), jnp.int32),
    )(dst2)

    _deg_kernel, _segsum_kernel = _sc_kernels()
    deg = _deg_kernel(dsth, ones_c, zeros_rd).reshape(NP, D)[:N]

    y1 = pl.pallas_call(
        _y1_body,
        grid=_grid,
        in_specs=[_row_spec, _full_spec, _row_spec],
        out_specs=_row_spec,
        out_shape=_nd_f32,
    )(x, W1, deg)

    p1 = _segsum_kernel(y1, src2, dsth, zeros_rd).reshape(NP, D)

    y2 = pl.pallas_call(
        _mid_body,
        grid=_grid,
        in_specs=[_row_spec, _row_spec, _row_spec, _bias_spec, _full_spec],
        out_specs=_row_spec,
        out_shape=_nd_f32,
    )(p1[:N], y1, deg, b1r, W2)

    p2 = _segsum_kernel(y2, src2, dsth, zeros_rd).reshape(NP, D)

    out = pl.pallas_call(
        _out_body,
        grid=_grid,
        in_specs=[_row_spec, _row_spec, _row_spec, _bias_spec],
        out_specs=_row_spec,
        out_shape=_nd_f32,
    )(p2[:N], y2, deg, b2r)
    return out


# speedup vs baseline: 14.0346x; 1.1234x over previous
import functools

import jax
import jax.numpy as jnp
from jax import lax
from jax.experimental import pallas as pl
from jax.experimental.pallas import tpu as pltpu
from jax.experimental.pallas import tpu_sc as plsc

N = 10000
D = 128
E = 320000
NC, NS = 2, 16
NW = NC * NS
C = 125
ERows = E // C
EPW = E // NW
GPW = EPW // 16
MAXCH = EPW // C
CAPC = 10256
NP = 10240
HALF = NP // 2
ACC_R = HALF + 8
SPT = HALF // NS
RB = 1000
NB = N // RB


def _compact_body(src_hbm, dst_hbm, fill0_hbm, fillt_hbm,
                  lsrc_hbm, ldst_hbm, cnts_hbm,
                  sv, dv, ls0, ld0, ls1, ld1, cv):
    cid = lax.axis_index("c")
    sid = lax.axis_index("s")
    wid = cid * NS + sid
    pltpu.sync_copy(src_hbm.at[pl.ds(wid * EPW, EPW)], sv)
    pltpu.sync_copy(dst_hbm.at[pl.ds(wid * EPW, EPW)], dv)
    pltpu.sync_copy(fill0_hbm, ls0)
    pltpu.sync_copy(fill0_hbm, ls1)
    pltpu.sync_copy(fillt_hbm, ld0)
    pltpu.sync_copy(fillt_hbm, ld1)

    lanes = lax.iota(jnp.int32, 16)
    garbage = CAPC - 16
    zero16 = jnp.zeros((16,), jnp.int32)

    def body(g, carry):
        n0v, n1v = carry
        s = sv[pl.ds(g * 16, 16)]
        d = dv[pl.ds(g * 16, 16)]
        m0 = d < HALF
        c0 = plsc.cumsum(m0.astype(jnp.int32))
        k0s = plsc.all_reduce_population_count(m0)
        pos0 = jnp.where(m0, n0v + c0 - 1, garbage + lanes)
        pos1 = jnp.where(m0, garbage + lanes, n1v + (lanes + 1 - c0) - 1)
        plsc.store_scatter(ls0, [pos0], s)
        plsc.store_scatter(ld0, [pos0], d)
        plsc.store_scatter(ls1, [pos1], s)
        plsc.store_scatter(ld1, [pos1], d - HALF)
        return n0v + k0s, n1v + (16 - k0s)

    n0v, n1v = lax.fori_loop(0, GPW, body, (zero16, zero16))
    pltpu.sync_copy(ls0, lsrc_hbm.at[wid, 0])
    pltpu.sync_copy(ld0, ldst_hbm.at[wid, 0])
    pltpu.sync_copy(ls1, lsrc_hbm.at[wid, 1])
    pltpu.sync_copy(ld1, ldst_hbm.at[wid, 1])
    cv[pl.ds(0, 16)] = n0v
    cv[pl.ds(16, 16)] = n1v
    pltpu.sync_copy(cv, cnts_hbm.at[wid])


def _valid(c, na, nb):
    return lax.select(c < MAXCH, c * C < na, (c - MAXCH) * C < nb)


def _deg_body(ldst_hbm, cnts_hbm, ones_hbm, zeros_hbm, degp_hbm,
              didx, ones_v, cv, acc):
    cid = lax.axis_index("c")
    sid = lax.axis_index("s")
    pltpu.sync_copy(zeros_hbm, acc.at[pl.ds(sid * SPT, SPT)])
    pltpu.sync_copy(ldst_hbm.at[2 * sid, cid], didx.at[pl.ds(0, MAXCH)])
    pltpu.sync_copy(ldst_hbm.at[2 * sid + 1, cid], didx.at[pl.ds(MAXCH, MAXCH)])
    pltpu.sync_copy(cnts_hbm.at[2 * sid, pl.ds(cid * 16, 16)],
                    cv.at[pl.ds(0, 16)])
    pltpu.sync_copy(cnts_hbm.at[2 * sid + 1, pl.ds(cid * 16, 16)],
                    cv.at[pl.ds(16, 16)])
    pltpu.sync_copy(ones_hbm, ones_v)
    plsc.subcore_barrier()
    na = cv[pl.ds(0, 16)][0]
    nb = cv[pl.ds(16, 16)][0]

    def body(c, _):
        @pl.when(_valid(c, na, nb))
        def _():
            pltpu.sync_copy(ones_v, acc.at[didx.at[c]], add=True)
        return 0

    lax.fori_loop(0, 2 * MAXCH, body, 0)
    plsc.subcore_barrier()
    pltpu.sync_copy(acc.at[pl.ds(sid * SPT, SPT)],
                    degp_hbm.at[cid, pl.ds(sid * SPT, SPT)])


def _segsum_body(y_hbm, lsrc_hbm, ldst_hbm, cnts_hbm, zeros_hbm, part_hbm,
                 sidx, didx, buf0, buf1, cv, acc, sem0, sem1, szero):
    cid = lax.axis_index("c")
    sid = lax.axis_index("s")
    dz = pltpu.async_copy(zeros_hbm, acc.at[pl.ds(sid * SPT, SPT)], szero)
    pltpu.sync_copy(lsrc_hbm.at[2 * sid, cid], sidx.at[pl.ds(0, MAXCH)])
    pltpu.sync_copy(lsrc_hbm.at[2 * sid + 1, cid], sidx.at[pl.ds(MAXCH, MAXCH)])
    pltpu.sync_copy(ldst_hbm.at[2 * sid, cid], didx.at[pl.ds(0, MAXCH)])
    pltpu.sync_copy(ldst_hbm.at[2 * sid + 1, cid], didx.at[pl.ds(MAXCH, MAXCH)])
    pltpu.sync_copy(cnts_hbm.at[2 * sid, pl.ds(cid * 16, 16)],
                    cv.at[pl.ds(0, 16)])
    pltpu.sync_copy(cnts_hbm.at[2 * sid + 1, pl.ds(cid * 16, 16)],
                    cv.at[pl.ds(16, 16)])
    dz.wait()
    plsc.subcore_barrier()
    na = cv[pl.ds(0, 16)][0]
    nb = cv[pl.ds(16, 16)][0]

    @pl.when(_valid(0, na, nb))
    def _():
        pltpu.async_copy(y_hbm.at[sidx.at[0]], buf0, sem0)

    def body(i, _):
        c0 = 2 * i
        v0 = _valid(c0, na, nb)
        v1 = _valid(c0 + 1, na, nb)
        v2 = _valid(c0 + 2, na, nb)

        @pl.when(v0)
        def _():
            pltpu.make_async_copy(y_hbm.at[sidx.at[c0]], buf0, sem0).wait()

        @pl.when(v1)
        def _():
            pltpu.async_copy(y_hbm.at[sidx.at[c0 + 1]], buf1, sem1)

        @pl.when(v0)
        def _():
            pltpu.sync_copy(buf0, acc.at[didx.at[c0]], add=True)

        @pl.when(v1)
        def _():
            pltpu.make_async_copy(y_hbm.at[sidx.at[c0 + 1]], buf1, sem1).wait()

        @pl.when(v2)
        def _():
            pltpu.async_copy(y_hbm.at[sidx.at[c0 + 2]], buf0, sem0)

        @pl.when(v1)
        def _():
            pltpu.sync_copy(buf1, acc.at[didx.at[c0 + 1]], add=True)

        return 0

    lax.fori_loop(0, MAXCH, body, 0)
    plsc.subcore_barrier()
    pltpu.sync_copy(acc.at[pl.ds(sid * SPT, SPT)],
                    part_hbm.at[cid, pl.ds(sid * SPT, SPT)])


@functools.cache
def _sc_kernels():
    mesh = plsc.VectorSubcoreMesh(core_axis_name="c", subcore_axis_name="s",
                                  num_cores=NC, num_subcores=NS)
    compact = pl.kernel(
        _compact_body,
        out_type=(
            jax.ShapeDtypeStruct((NW, 2, CAPC), jnp.int32),
            jax.ShapeDtypeStruct((NW, 2, CAPC), jnp.int32),
            jax.ShapeDtypeStruct((NW, 32), jnp.int32),
        ),
        mesh=mesh,
        compiler_params=pltpu.CompilerParams(needs_layout_passes=False),
        scratch_types=[
            pltpu.VMEM((EPW,), jnp.int32),
            pltpu.VMEM((EPW,), jnp.int32),
            pltpu.VMEM((CAPC,), jnp.int32),
            pltpu.VMEM((CAPC,), jnp.int32),
            pltpu.VMEM((CAPC,), jnp.int32),
            pltpu.VMEM((CAPC,), jnp.int32),
            pltpu.VMEM((32,), jnp.int32),
        ],
    )
    deg = pl.kernel(
        _deg_body,
        out_type=jax.ShapeDtypeStruct((NC, HALF, D), jnp.float32),
        mesh=mesh,
        scratch_types=[
            pltpu.VMEM((2 * MAXCH, C), jnp.int32),
            pltpu.VMEM((C, D), jnp.float32),
            pltpu.VMEM((32,), jnp.int32),
            pltpu.VMEM_SHARED((ACC_R, D), jnp.float32),
        ],
    )
    segsum = pl.kernel(
        _segsum_body,
        out_type=jax.ShapeDtypeStruct((NC, HALF, D), jnp.float32),
        mesh=mesh,
        scratch_types=[
            pltpu.VMEM((2 * MAXCH, C), jnp.int32),
            pltpu.VMEM((2 * MAXCH, C), jnp.int32),
            pltpu.VMEM((C, D), jnp.float32),
            pltpu.VMEM((C, D), jnp.float32),
            pltpu.VMEM((32,), jnp.int32),
            pltpu.VMEM_SHARED((ACC_R, D), jnp.float32),
            pltpu.SemaphoreType.DMA,
            pltpu.SemaphoreType.DMA,
            pltpu.SemaphoreType.DMA,
        ],
    )
    return compact, deg, segsum


def _dinv(deg_ref):
    return lax.rsqrt(deg_ref[:, :1] + 1.0)


def _y1_body(x_ref, w_ref, deg_ref, y_ref):
    y_ref[...] = _dinv(deg_ref) * jnp.dot(
        x_ref[...], w_ref[...], preferred_element_type=jnp.float32)


def _mid_body(p_ref, y1_ref, deg_ref, b1_ref, w2_ref, y2_ref):
    dinv = _dinv(deg_ref)
    h = jnp.maximum(dinv * (p_ref[...] + y1_ref[...]) + b1_ref[...], 0.0)
    y2_ref[...] = dinv * jnp.dot(h, w2_ref[...],
                                 preferred_element_type=jnp.float32)


def _out_body(p_ref, y2_ref, deg_ref, b2_ref, o_ref):
    dinv = _dinv(deg_ref)
    o_ref[...] = dinv * (p_ref[...] + y2_ref[...]) + b2_ref[...]


_row_spec = pl.BlockSpec((RB, D), lambda i: (i, 0))
_full_spec = pl.BlockSpec((D, D), lambda i: (0, 0))
_bias_spec = pl.BlockSpec((1, D), lambda i: (0, 0))
_grid = (NB,)
_nd_f32 = jax.ShapeDtypeStruct((N, D), jnp.float32)


def kernel(x, edge_index, W1, b1, W2, b2):
    src1 = edge_index[0].astype(jnp.int32)
    dst1 = edge_index[1].astype(jnp.int32)
    ones_c = jnp.ones((C, D), jnp.float32)
    zeros_rd = jnp.zeros((SPT, D), jnp.float32)
    fill0 = jnp.zeros((CAPC,), jnp.int32)
    fillt = jnp.full((CAPC,), HALF, jnp.int32)
    b1r = b1.reshape(1, D)
    b2r = b2.reshape(1, D)

    _compact_kernel, _deg_kernel, _segsum_kernel = _sc_kernels()
    lsrc, ldst, cnts = _compact_kernel(src1, dst1, fill0, fillt)
    lsrc = lsrc[:, :, :EPW].reshape(NW, 2, MAXCH, C)
    ldst = ldst[:, :, :EPW].reshape(NW, 2, MAXCH, C)

    deg = _deg_kernel(ldst, cnts, ones_c, zeros_rd).reshape(NP, D)[:N]

    y1 = pl.pallas_call(
        _y1_body,
        grid=_grid,
        in_specs=[_row_spec, _full_spec, _row_spec],
        out_specs=_row_spec,
        out_shape=_nd_f32,
    )(x, W1, deg)

    p1 = _segsum_kernel(y1, lsrc, ldst, cnts, zeros_rd).reshape(NP, D)

    y2 = pl.pallas_call(
        _mid_body,
        grid=_grid,
        in_specs=[_row_spec, _row_spec, _row_spec, _bias_spec, _full_spec],
        out_specs=_row_spec,
        out_shape=_nd_f32,
    )(p1[:N], y1, deg, b1r, W2)

    p2 = _segsum_kernel(y2, lsrc, ldst, cnts, zeros_rd).reshape(NP, D)

    out = pl.pallas_call(
        _out_body,
        grid=_grid,
        in_specs=[_row_spec, _row_spec, _row_spec, _bias_spec],
        out_specs=_row_spec,
        out_shape=_nd_f32,
    )(p2[:N], y2, deg, b2r)
    return out


# speedup vs baseline: 16.6231x; 1.1844x over previous
import functools

import jax
import jax.numpy as jnp
from jax import lax
from jax.experimental import pallas as pl
from jax.experimental.pallas import tpu as pltpu
from jax.experimental.pallas import tpu_sc as plsc

N = 10000
D = 128
E = 320000
NC, NS = 2, 16
NW = NC * NS
C = 125
ERows = E // C
EPW = E // NW
GPW = EPW // 16
MAXCH = EPW // C
CAPC = 10160
NP = 10240
HALF = NP // 2
BOUND = 5096
TRASH = 5100
ACC_R = 5104
SPT = 320
SPT15 = BOUND - 15 * SPT
RB = 1000
NB = N // RB


def _compact_body(src_hbm, dst_hbm, fill0_hbm, fillt_hbm, zeros_np_hbm,
                  lsrc_hbm, ldst_hbm, cnts_hbm, hists_hbm,
                  sv, dv, ls0, ld0, ls1, ld1, cv, hist):
    cid = lax.axis_index("c")
    sid = lax.axis_index("s")
    wid = cid * NS + sid
    pltpu.sync_copy(src_hbm.at[pl.ds(wid * EPW, EPW)], sv)
    pltpu.sync_copy(dst_hbm.at[pl.ds(wid * EPW, EPW)], dv)
    pltpu.sync_copy(fill0_hbm, ls0)
    pltpu.sync_copy(fill0_hbm, ls1)
    pltpu.sync_copy(fillt_hbm, ld0)
    pltpu.sync_copy(fillt_hbm, ld1)
    pltpu.sync_copy(zeros_np_hbm, hist)

    lanes = lax.iota(jnp.int32, 16)
    garbage = CAPC - 16
    zero16 = jnp.zeros((16,), jnp.int32)
    onesf = jnp.ones((16,), jnp.float32)

    def body(g, carry):
        n0v, n1v = carry
        s = sv[pl.ds(g * 16, 16)]
        d = dv[pl.ds(g * 16, 16)]
        m0 = d < BOUND
        c0 = plsc.cumsum(m0.astype(jnp.int32))
        k0s = plsc.all_reduce_population_count(m0)
        pos0 = jnp.where(m0, n0v + c0 - 1, garbage + lanes)
        pos1 = jnp.where(m0, garbage + lanes, n1v + (lanes + 1 - c0) - 1)
        plsc.store_scatter(ls0, [pos0], s)
        plsc.store_scatter(ld0, [pos0], d)
        plsc.store_scatter(ls1, [pos1], s)
        plsc.store_scatter(ld1, [pos1], d - BOUND)
        plsc.addupdate_scatter(hist, [d], onesf)
        return n0v + k0s, n1v + (16 - k0s)

    n0v, n1v = lax.fori_loop(0, GPW, body, (zero16, zero16))
    pltpu.sync_copy(ls0, lsrc_hbm.at[wid, 0])
    pltpu.sync_copy(ld0, ldst_hbm.at[wid, 0])
    pltpu.sync_copy(ls1, lsrc_hbm.at[wid, 1])
    pltpu.sync_copy(ld1, ldst_hbm.at[wid, 1])
    pltpu.sync_copy(hist, hists_hbm.at[wid])
    cv[pl.ds(0, 16)] = n0v
    cv[pl.ds(16, 16)] = n1v
    pltpu.sync_copy(cv, cnts_hbm.at[wid])


def _valid(c, na, nb):
    return lax.select(c < MAXCH, c * C < na, (c - MAXCH) * C < nb)


def _segsum_body(y_hbm, lsrc_hbm, ldst_hbm, cnts_hbm, zeros_hbm, part_hbm,
                 sidx, didx, buf0, buf1, buf2, cv, acc,
                 sem0, sem1, sem2, szero):
    bufs = (buf0, buf1, buf2)
    sems = (sem0, sem1, sem2)
    cid = lax.axis_index("c")
    sid = lax.axis_index("s")
    @pl.when(sid < 15)
    def _():
        pltpu.async_copy(zeros_hbm, acc.at[pl.ds(sid * SPT, SPT)], szero)

    @pl.when(sid == 15)
    def _():
        pltpu.async_copy(zeros_hbm.at[pl.ds(0, SPT15)],
                         acc.at[pl.ds(15 * SPT, SPT15)], szero)
    pltpu.sync_copy(lsrc_hbm.at[2 * sid, cid], sidx.at[pl.ds(0, MAXCH)])
    pltpu.sync_copy(lsrc_hbm.at[2 * sid + 1, cid], sidx.at[pl.ds(MAXCH, MAXCH)])
    pltpu.sync_copy(ldst_hbm.at[2 * sid, cid], didx.at[pl.ds(0, MAXCH)])
    pltpu.sync_copy(ldst_hbm.at[2 * sid + 1, cid], didx.at[pl.ds(MAXCH, MAXCH)])
    pltpu.sync_copy(cnts_hbm.at[2 * sid, pl.ds(cid * 16, 16)],
                    cv.at[pl.ds(0, 16)])
    pltpu.sync_copy(cnts_hbm.at[2 * sid + 1, pl.ds(cid * 16, 16)],
                    cv.at[pl.ds(16, 16)])
    @pl.when(sid < 15)
    def _():
        pltpu.make_async_copy(zeros_hbm, acc.at[pl.ds(0, SPT)], szero).wait()

    @pl.when(sid == 15)
    def _():
        pltpu.make_async_copy(zeros_hbm.at[pl.ds(0, SPT15)],
                              acc.at[pl.ds(0, SPT15)], szero).wait()

    plsc.subcore_barrier()
    na = cv[pl.ds(0, 16)][0]
    nb = cv[pl.ds(16, 16)][0]

    for c in range(2):
        @pl.when(_valid(c, na, nb))
        def _(c=c):
            pltpu.async_copy(y_hbm.at[sidx.at[c]], bufs[c], sems[c])

    def body(i, _):
        c0 = 3 * i
        for j in range(3):
            c = c0 + j
            vc = _valid(c, na, nb)
            vn = _valid(c + 2, na, nb)

            @pl.when(vc)
            def _(c=c, j=j):
                pltpu.make_async_copy(y_hbm.at[sidx.at[c]], bufs[j],
                                      sems[j]).wait()

            @pl.when(vn)
            def _(c=c, j=j):
                pltpu.async_copy(y_hbm.at[sidx.at[c + 2]], bufs[(j + 2) % 3],
                                 sems[(j + 2) % 3])

            @pl.when(vc)
            def _(c=c, j=j):
                pltpu.sync_copy(bufs[j], acc.at[didx.at[c]], add=True)

        return 0

    lax.fori_loop(0, (2 * MAXCH + 2) // 3, body, 0)
    plsc.subcore_barrier()

    @pl.when(sid < 15)
    def _():
        pltpu.sync_copy(acc.at[pl.ds(sid * SPT, SPT)],
                        part_hbm.at[cid, pl.ds(sid * SPT, SPT)])

    @pl.when(sid == 15)
    def _():
        pltpu.sync_copy(acc.at[pl.ds(15 * SPT, SPT15)],
                        part_hbm.at[cid, pl.ds(15 * SPT, SPT15)])


@functools.cache
def _sc_kernels():
    mesh = plsc.VectorSubcoreMesh(core_axis_name="c", subcore_axis_name="s",
                                  num_cores=NC, num_subcores=NS)
    compact = pl.kernel(
        _compact_body,
        out_type=(
            jax.ShapeDtypeStruct((NW, 2, CAPC), jnp.int32),
            jax.ShapeDtypeStruct((NW, 2, CAPC), jnp.int32),
            jax.ShapeDtypeStruct((NW, 32), jnp.int32),
            jax.ShapeDtypeStruct((NW, NP), jnp.float32),
        ),
        mesh=mesh,
        compiler_params=pltpu.CompilerParams(needs_layout_passes=False),
        scratch_types=[
            pltpu.VMEM((EPW,), jnp.int32),
            pltpu.VMEM((EPW,), jnp.int32),
            pltpu.VMEM((CAPC,), jnp.int32),
            pltpu.VMEM((CAPC,), jnp.int32),
            pltpu.VMEM((CAPC,), jnp.int32),
            pltpu.VMEM((CAPC,), jnp.int32),
            pltpu.VMEM((32,), jnp.int32),
            pltpu.VMEM((NP,), jnp.float32),
        ],
    )
    segsum = pl.kernel(
        _segsum_body,
        out_type=jax.ShapeDtypeStruct((NC, BOUND, D), jnp.float32),
        mesh=mesh,
        scratch_types=[
            pltpu.VMEM((2 * MAXCH, C), jnp.int32),
            pltpu.VMEM((2 * MAXCH, C), jnp.int32),
            pltpu.VMEM((C, D), jnp.float32),
            pltpu.VMEM((C, D), jnp.float32),
            pltpu.VMEM((C, D), jnp.float32),
            pltpu.VMEM((32,), jnp.int32),
            pltpu.VMEM_SHARED((ACC_R, D), jnp.float32),
            pltpu.SemaphoreType.DMA,
            pltpu.SemaphoreType.DMA,
            pltpu.SemaphoreType.DMA,
            pltpu.SemaphoreType.DMA,
        ],
    )
    return compact, segsum


def _degsum_body(h_ref, deg_ref):
    deg_ref[...] = jnp.sum(h_ref[...], axis=0)[:, None]


def _dinv(deg_ref):
    return lax.rsqrt(deg_ref[...] + 1.0)


def _y1_body(x_ref, w_ref, deg_ref, y_ref):
    y_ref[...] = _dinv(deg_ref) * jnp.dot(
        x_ref[...], w_ref[...], preferred_element_type=jnp.float32)


def _mid_body(p_ref, y1_ref, deg_ref, b1_ref, w2_ref, y2_ref):
    dinv = _dinv(deg_ref)
    h = jnp.maximum(dinv * (p_ref[...] + y1_ref[...]) + b1_ref[...], 0.0)
    y2_ref[...] = dinv * jnp.dot(h, w2_ref[...],
                                 preferred_element_type=jnp.float32)


def _out_body(p_ref, y2_ref, deg_ref, b2_ref, o_ref):
    dinv = _dinv(deg_ref)
    o_ref[...] = dinv * (p_ref[...] + y2_ref[...]) + b2_ref[...]


_row_spec = pl.BlockSpec((RB, D), lambda i: (i, 0))
_full_spec = pl.BlockSpec((D, D), lambda i: (0, 0))
_bias_spec = pl.BlockSpec((1, D), lambda i: (0, 0))
_deg_spec = pl.BlockSpec((RB, 1), lambda i: (i, 0))
_grid = (NB,)
_nd_f32 = jax.ShapeDtypeStruct((N, D), jnp.float32)
DSB = 1280


def kernel(x, edge_index, W1, b1, W2, b2):
    src1 = edge_index[0].astype(jnp.int32)
    dst1 = edge_index[1].astype(jnp.int32)
    zeros_rd = jnp.zeros((SPT, D), jnp.float32)
    zeros_np = jnp.zeros((NP,), jnp.float32)
    fill0 = jnp.zeros((CAPC,), jnp.int32)
    fillt = jnp.full((CAPC,), TRASH, jnp.int32)
    b1r = b1.reshape(1, D)
    b2r = b2.reshape(1, D)

    _compact_kernel, _segsum_kernel = _sc_kernels()
    lsrc, ldst, cnts, hists = _compact_kernel(src1, dst1, fill0, fillt,
                                              zeros_np)
    lsrc = lsrc[:, :, :EPW].reshape(NW, 2, MAXCH, C)
    ldst = ldst[:, :, :EPW].reshape(NW, 2, MAXCH, C)

    deg = pl.pallas_call(
        _degsum_body,
        grid=(NP // DSB,),
        in_specs=[pl.BlockSpec((NW, DSB), lambda i: (0, i))],
        out_specs=pl.BlockSpec((DSB, 1), lambda i: (i, 0)),
        out_shape=jax.ShapeDtypeStruct((NP, 1), jnp.float32),
    )(hists)[:N]

    y1 = pl.pallas_call(
        _y1_body,
        grid=_grid,
        in_specs=[_row_spec, _full_spec, _deg_spec],
        out_specs=_row_spec,
        out_shape=_nd_f32,
    )(x, W1, deg)

    p1 = _segsum_kernel(y1, lsrc, ldst, cnts, zeros_rd)
    p1 = jnp.concatenate([p1[0, :BOUND], p1[1, :N - BOUND]])

    y2 = pl.pallas_call(
        _mid_body,
        grid=_grid,
        in_specs=[_row_spec, _row_spec, _deg_spec, _bias_spec, _full_spec],
        out_specs=_row_spec,
        out_shape=_nd_f32,
    )(p1, y1, deg, b1r, W2)

    p2 = _segsum_kernel(y2, lsrc, ldst, cnts, zeros_rd)
    p2 = jnp.concatenate([p2[0, :BOUND], p2[1, :N - BOUND]])

    out = pl.pallas_call(
        _out_body,
        grid=_grid,
        in_specs=[_row_spec, _row_spec, _deg_spec, _bias_spec],
        out_specs=_row_spec,
        out_shape=_nd_f32,
    )(p2, y2, deg, b2r)
    return out


# speedup vs baseline: 16.9029x; 1.0168x over previous
import functools

import jax
import jax.numpy as jnp
from jax import lax
from jax.experimental import pallas as pl
from jax.experimental.pallas import tpu as pltpu
from jax.experimental.pallas import tpu_sc as plsc

N = 10000
D = 128
E = 320000
NC, NS = 2, 16
NW = NC * NS
C = 125
ERows = E // C
EPW = E // NW
GPW = EPW // 16
MAXCH = EPW // C
CAPC = 10160
NP = 10240
HALF = NP // 2
BOUND = 5096
TRASH = 5100
ACC_R = 5104
SPT = 320
SPT15 = BOUND - 15 * SPT
RB = 1000
NB = N // RB


def _compact_body(src_hbm, dst_hbm, fill0_hbm, fillt_hbm, zeros_np_hbm,
                  lsrc_hbm, ldst_hbm, cnts_hbm, hists_hbm,
                  sv, dv, ls0, ld0, ls1, ld1, cv, hist):
    cid = lax.axis_index("c")
    sid = lax.axis_index("s")
    wid = cid * NS + sid
    pltpu.sync_copy(src_hbm.at[pl.ds(wid * EPW, EPW)], sv)
    pltpu.sync_copy(dst_hbm.at[pl.ds(wid * EPW, EPW)], dv)
    pltpu.sync_copy(fill0_hbm, ls0)
    pltpu.sync_copy(fill0_hbm, ls1)
    pltpu.sync_copy(fillt_hbm, ld0)
    pltpu.sync_copy(fillt_hbm, ld1)
    pltpu.sync_copy(zeros_np_hbm, hist)

    lanes = lax.iota(jnp.int32, 16)
    garbage = CAPC - 16
    zero16 = jnp.zeros((16,), jnp.int32)
    onesf = jnp.ones((16,), jnp.float32)

    def body(g, carry):
        n0v, n1v = carry
        s = sv[pl.ds(g * 16, 16)]
        d = dv[pl.ds(g * 16, 16)]
        m0 = d < BOUND
        c0 = plsc.cumsum(m0.astype(jnp.int32))
        k0s = plsc.all_reduce_population_count(m0)
        pos0 = jnp.where(m0, n0v + c0 - 1, garbage + lanes)
        pos1 = jnp.where(m0, garbage + lanes, n1v + (lanes + 1 - c0) - 1)
        plsc.store_scatter(ls0, [pos0], s)
        plsc.store_scatter(ld0, [pos0], d)
        plsc.store_scatter(ls1, [pos1], s)
        plsc.store_scatter(ld1, [pos1], d - BOUND)
        plsc.addupdate_scatter(hist, [d], onesf)
        return n0v + k0s, n1v + (16 - k0s)

    n0v, n1v = lax.fori_loop(0, GPW, body, (zero16, zero16))
    pltpu.sync_copy(ls0, lsrc_hbm.at[wid, 0])
    pltpu.sync_copy(ld0, ldst_hbm.at[wid, 0])
    pltpu.sync_copy(ls1, lsrc_hbm.at[wid, 1])
    pltpu.sync_copy(ld1, ldst_hbm.at[wid, 1])
    pltpu.sync_copy(hist, hists_hbm.at[wid])
    cv[pl.ds(0, 16)] = n0v
    cv[pl.ds(16, 16)] = n1v
    pltpu.sync_copy(cv, cnts_hbm.at[wid])


def _valid(c, na, nb):
    return lax.select(c < MAXCH, c * C < na, (c - MAXCH) * C < nb)


def _segsum_body(y_hbm, lsrc_hbm, ldst_hbm, cnts_hbm, zeros_hbm, part_hbm,
                 sidx, didx, buf0, buf1, buf2, cv, acc,
                 sem0, sem1, sem2, ssem0, ssem1, ssem2, szero):
    bufs = (buf0, buf1, buf2)
    sems = (sem0, sem1, sem2)
    ssems = (ssem0, ssem1, ssem2)
    cid = lax.axis_index("c")
    sid = lax.axis_index("s")
    @pl.when(sid < 15)
    def _():
        pltpu.async_copy(zeros_hbm, acc.at[pl.ds(sid * SPT, SPT)], szero)

    @pl.when(sid == 15)
    def _():
        pltpu.async_copy(zeros_hbm.at[pl.ds(0, SPT15)],
                         acc.at[pl.ds(15 * SPT, SPT15)], szero)
    pltpu.sync_copy(lsrc_hbm.at[2 * sid, cid], sidx.at[pl.ds(0, MAXCH)])
    pltpu.sync_copy(lsrc_hbm.at[2 * sid + 1, cid], sidx.at[pl.ds(MAXCH, MAXCH)])
    pltpu.sync_copy(ldst_hbm.at[2 * sid, cid], didx.at[pl.ds(0, MAXCH)])
    pltpu.sync_copy(ldst_hbm.at[2 * sid + 1, cid], didx.at[pl.ds(MAXCH, MAXCH)])
    pltpu.sync_copy(cnts_hbm.at[2 * sid, pl.ds(cid * 16, 16)],
                    cv.at[pl.ds(0, 16)])
    pltpu.sync_copy(cnts_hbm.at[2 * sid + 1, pl.ds(cid * 16, 16)],
                    cv.at[pl.ds(16, 16)])
    @pl.when(sid < 15)
    def _():
        pltpu.make_async_copy(zeros_hbm, acc.at[pl.ds(0, SPT)], szero).wait()

    @pl.when(sid == 15)
    def _():
        pltpu.make_async_copy(zeros_hbm.at[pl.ds(0, SPT15)],
                              acc.at[pl.ds(0, SPT15)], szero).wait()

    plsc.subcore_barrier()
    na = cv[pl.ds(0, 16)][0]
    nb = cv[pl.ds(16, 16)][0]

    for c in range(2):
        @pl.when(_valid(c, na, nb))
        def _(c=c):
            pltpu.async_copy(y_hbm.at[sidx.at[c]], bufs[c], sems[c])

    def body(i, _):
        c0 = 3 * i
        for j in range(3):
            c = c0 + j
            j2 = (j + 2) % 3
            vc = _valid(c, na, nb)
            cm = jnp.maximum(c - 1, 0)
            cp = jnp.minimum(c + 2, 2 * MAXCH - 1)

            @pl.when((c >= 1) & _valid(c - 1, na, nb))
            def _(cm=cm, j2=j2):
                pltpu.make_async_copy(bufs[j2], acc.at[didx.at[cm]],
                                      ssems[j2]).wait()

            @pl.when(_valid(c + 2, na, nb))
            def _(cp=cp, j2=j2):
                pltpu.async_copy(y_hbm.at[sidx.at[cp]], bufs[j2], sems[j2])

            @pl.when(vc)
            def _(c=c, j=j):
                pltpu.make_async_copy(y_hbm.at[sidx.at[c]], bufs[j],
                                      sems[j]).wait()

            @pl.when(vc)
            def _(c=c, j=j):
                pltpu.async_copy(bufs[j], acc.at[didx.at[c]], ssems[j])

        return 0

    lax.fori_loop(0, (2 * MAXCH + 2) // 3, body, 0)
    plsc.subcore_barrier()

    @pl.when(sid < 15)
    def _():
        pltpu.sync_copy(acc.at[pl.ds(sid * SPT, SPT)],
                        part_hbm.at[cid, pl.ds(sid * SPT, SPT)])

    @pl.when(sid == 15)
    def _():
        pltpu.sync_copy(acc.at[pl.ds(15 * SPT, SPT15)],
                        part_hbm.at[cid, pl.ds(15 * SPT, SPT15)])


@functools.cache
def _sc_kernels():
    mesh = plsc.VectorSubcoreMesh(core_axis_name="c", subcore_axis_name="s",
                                  num_cores=NC, num_subcores=NS)
    compact = pl.kernel(
        _compact_body,
        out_type=(
            jax.ShapeDtypeStruct((NW, 2, CAPC), jnp.int32),
            jax.ShapeDtypeStruct((NW, 2, CAPC), jnp.int32),
            jax.ShapeDtypeStruct((NW, 32), jnp.int32),
            jax.ShapeDtypeStruct((NW, NP), jnp.float32),
        ),
        mesh=mesh,
        compiler_params=pltpu.CompilerParams(needs_layout_passes=False),
        scratch_types=[
            pltpu.VMEM((EPW,), jnp.int32),
            pltpu.VMEM((EPW,), jnp.int32),
            pltpu.VMEM((CAPC,), jnp.int32),
            pltpu.VMEM((CAPC,), jnp.int32),
            pltpu.VMEM((CAPC,), jnp.int32),
            pltpu.VMEM((CAPC,), jnp.int32),
            pltpu.VMEM((32,), jnp.int32),
            pltpu.VMEM((NP,), jnp.float32),
        ],
    )
    segsum = pl.kernel(
        _segsum_body,
        out_type=jax.ShapeDtypeStruct((NC, BOUND, D), jnp.float32),
        mesh=mesh,
        scratch_types=[
            pltpu.VMEM((2 * MAXCH, C), jnp.int32),
            pltpu.VMEM((2 * MAXCH, C), jnp.int32),
            pltpu.VMEM((C, D), jnp.float32),
            pltpu.VMEM((C, D), jnp.float32),
            pltpu.VMEM((C, D), jnp.float32),
            pltpu.VMEM((32,), jnp.int32),
            pltpu.VMEM_SHARED((ACC_R, D), jnp.float32),
            pltpu.SemaphoreType.DMA,
            pltpu.SemaphoreType.DMA,
            pltpu.SemaphoreType.DMA,
            pltpu.SemaphoreType.DMA,
            pltpu.SemaphoreType.DMA,
            pltpu.SemaphoreType.DMA,
            pltpu.SemaphoreType.DMA,
        ],
    )
    return compact, segsum


def _degsum_body(h_ref, deg_ref):
    deg_ref[...] = jnp.sum(h_ref[...], axis=0)[:, None]


def _dinv(deg_ref):
    return lax.rsqrt(deg_ref[...] + 1.0)


def _y1_body(x_ref, w_ref, deg_ref, y_ref):
    y_ref[...] = _dinv(deg_ref) * jnp.dot(
        x_ref[...], w_ref[...], preferred_element_type=jnp.float32)


def _mid_body(p_ref, y1_ref, deg_ref, b1_ref, w2_ref, y2_ref):
    dinv = _dinv(deg_ref)
    h = jnp.maximum(dinv * (p_ref[...] + y1_ref[...]) + b1_ref[...], 0.0)
    y2_ref[...] = dinv * jnp.dot(h, w2_ref[...],
                                 preferred_element_type=jnp.float32)


def _out_body(p_ref, y2_ref, deg_ref, b2_ref, o_ref):
    dinv = _dinv(deg_ref)
    o_ref[...] = dinv * (p_ref[...] + y2_ref[...]) + b2_ref[...]


_row_spec = pl.BlockSpec((RB, D), lambda i: (i, 0))
_full_spec = pl.BlockSpec((D, D), lambda i: (0, 0))
_bias_spec = pl.BlockSpec((1, D), lambda i: (0, 0))
_deg_spec = pl.BlockSpec((RB, 1), lambda i: (i, 0))
_grid = (NB,)
_nd_f32 = jax.ShapeDtypeStruct((N, D), jnp.float32)
DSB = 1280


def kernel(x, edge_index, W1, b1, W2, b2):
    src1 = edge_index[0].astype(jnp.int32)
    dst1 = edge_index[1].astype(jnp.int32)
    zeros_rd = jnp.zeros((SPT, D), jnp.float32)
    zeros_np = jnp.zeros((NP,), jnp.float32)
    fill0 = jnp.zeros((CAPC,), jnp.int32)
    fillt = jnp.full((CAPC,), TRASH, jnp.int32)
    b1r = b1.reshape(1, D)
    b2r = b2.reshape(1, D)

    _compact_kernel, _segsum_kernel = _sc_kernels()
    lsrc, ldst, cnts, hists = _compact_kernel(src1, dst1, fill0, fillt,
                                              zeros_np)
    lsrc = lsrc[:, :, :EPW].reshape(NW, 2, MAXCH, C)
    ldst = ldst[:, :, :EPW].reshape(NW, 2, MAXCH, C)

    deg = pl.pallas_call(
        _degsum_body,
        grid=(NP // DSB,),
        in_specs=[pl.BlockSpec((NW, DSB), lambda i: (0, i))],
        out_specs=pl.BlockSpec((DSB, 1), lambda i: (i, 0)),
        out_shape=jax.ShapeDtypeStruct((NP, 1), jnp.float32),
    )(hists)[:N]

    y1 = pl.pallas_call(
        _y1_body,
        grid=_grid,
        in_specs=[_row_spec, _full_spec, _deg_spec],
        out_specs=_row_spec,
        out_shape=_nd_f32,
    )(x, W1, deg)

    p1 = _segsum_kernel(y1, lsrc, ldst, cnts, zeros_rd)
    p1 = jnp.concatenate([p1[0, :BOUND], p1[1, :N - BOUND]])

    y2 = pl.pallas_call(
        _mid_body,
        grid=_grid,
        in_specs=[_row_spec, _row_spec, _deg_spec, _bias_spec, _full_spec],
        out_specs=_row_spec,
        out_shape=_nd_f32,
    )(p1, y1, deg, b1r, W2)

    p2 = _segsum_kernel(y2, lsrc, ldst, cnts, zeros_rd)
    p2 = jnp.concatenate([p2[0, :BOUND], p2[1, :N - BOUND]])

    out = pl.pallas_call(
        _out_body,
        grid=_grid,
        in_specs=[_row_spec, _row_spec, _deg_spec, _bias_spec],
        out_specs=_row_spec,
        out_shape=_nd_f32,
    )(p2, y2, deg, b2r)
    return out
